# baseline (speedup 1.0000x reference)
import functools

import jax
import jax.numpy as jnp
from jax import lax
from jax.experimental import pallas as pl
from jax.experimental.pallas import tpu as pltpu
from jax.experimental.pallas import tpu_sc as plsc

H = 128
GROUPS = 16
PER_GROUP = 8
SLOTS = GROUPS * PER_GROUP
B = 4096
ROW = SLOTS * H
OUT = GROUPS * H

BT = 2560
BSC = B - BT
BR = 128
NBLK = BT // BR

NC, NS, L = 2, 16, 16
NW = NC * NS
ROWS_PER_W = BSC // NW
CR = 8
NQ = 4
CC = ROW // NQ
NGR = ROWS_PER_W // CR

_MESH = plsc.VectorSubcoreMesh(core_axis_name="c", subcore_axis_name="s")


def _fold_chunk(buf, acc):
    @pl.loop(0, GROUPS)
    def _group(i):
        for hp in range(H // L):
            off = i * H + hp * L
            a = acc[pl.ds(off, L)]
            for r in range(CR):
                row = buf.at[r]
                for jj in range(CC // (GROUPS * H)):
                    a = jnp.maximum(a, row[pl.ds(jj * GROUPS * H + off, L)])
            acc[pl.ds(off, L)] = a


@functools.partial(
    pl.kernel,
    out_type=jax.ShapeDtypeStruct((NW, OUT), jnp.float32),
    mesh=_MESH,
    scratch_types=[
        pltpu.VMEM((CR, CC), jnp.float32),
        pltpu.VMEM((CR, CC), jnp.float32),
        pltpu.VMEM((OUT,), jnp.float32),
        pltpu.SemaphoreType.DMA,
        pltpu.SemaphoreType.DMA,
    ],
)
def _sc_stage(x_hbm, part_hbm, buf0, buf1, acc, sem0, sem1):
    wid = lax.axis_index("s") * NC + lax.axis_index("c")
    row0 = BT + wid * ROWS_PER_W
    bufs = (buf0, buf1)
    sems = (sem0, sem1)

    neg = jnp.full((L,), -jnp.inf, jnp.float32)

    @pl.loop(0, OUT // L)
    def _init(p):
        acc[pl.ds(p * L, L)] = neg

    def _start(gr, q, b):
        pltpu.async_copy(
            x_hbm.at[pl.ds(row0 + gr * CR, CR), pl.ds(q * CC, CC)],
            bufs[b],
            sems[b],
        )

    def _wait(b):
        pltpu.make_async_copy(
            x_hbm.at[pl.ds(0, CR), pl.ds(0, CC)], bufs[b], sems[b]
        ).wait()

    _start(0, 0, 0)

    @pl.loop(0, NGR)
    def _main(gr):
        for q in range(NQ):
            b = q % 2
            nb = (q + 1) % 2
            if q < NQ - 1:
                _start(gr, q + 1, nb)
            else:

                @pl.when(gr + 1 < NGR)
                def _prefetch():
                    _start(gr + 1, 0, nb)

            _wait(b)
            _fold_chunk(bufs[b], acc)

    pltpu.sync_copy(acc, part_hbm.at[wid])


def _tc_body(x_ref, o_ref):
    o_ref[...] = jnp.max(x_ref[...], axis=0, keepdims=True)[None]


def _combine_body(sc_ref, tc_ref, o_ref):
    t = jnp.max(tc_ref[...].reshape(NBLK, ROW), axis=0)
    t = jnp.max(t.reshape(PER_GROUP, GROUPS, H), axis=0)
    s = jnp.max(sc_ref[...], axis=0).reshape(GROUPS, H)
    o_ref[...] = jnp.maximum(t, s).reshape(1, OUT)


def kernel(x):
    sc_parts = _sc_stage(x)
    tc_parts = pl.pallas_call(
        _tc_body,
        grid=(NBLK,),
        in_specs=[pl.BlockSpec((BR, ROW), lambda i: (i, 0))],
        out_specs=pl.BlockSpec((1, 1, ROW), lambda i: (i, 0, 0)),
        out_shape=jax.ShapeDtypeStruct((NBLK, 1, ROW), jnp.float32),
    )(x)
    return pl.pallas_call(
        _combine_body,
        out_shape=jax.ShapeDtypeStruct((1, OUT), jnp.float32),
    )(sc_parts, tc_parts)


# speedup vs baseline: 1.0133x; 1.0133x over previous
import functools

import jax
import jax.numpy as jnp
from jax import lax
from jax.experimental import pallas as pl
from jax.experimental.pallas import tpu as pltpu
from jax.experimental.pallas import tpu_sc as plsc

H = 128
GROUPS = 16
PER_GROUP = 8
SLOTS = GROUPS * PER_GROUP
B = 4096
ROW = SLOTS * H
OUT = GROUPS * H

BT = 3584
BSC = B - BT
BR = 128
NBLK = BT // BR

NC, NS, L = 2, 16, 16
NW = NC * NS
ROWS_PER_W = BSC // NW
CR = 8
NQ = 4
CC = ROW // NQ
NGR = ROWS_PER_W // CR

_MESH = plsc.VectorSubcoreMesh(core_axis_name="c", subcore_axis_name="s")


def _fold_chunk(buf, acc):
    @pl.loop(0, GROUPS)
    def _group(i):
        for hp in range(H // L):
            off = i * H + hp * L
            a = acc[pl.ds(off, L)]
            for r in range(CR):
                row = buf.at[r]
                for jj in range(CC // (GROUPS * H)):
                    a = jnp.maximum(a, row[pl.ds(jj * GROUPS * H + off, L)])
            acc[pl.ds(off, L)] = a


@functools.partial(
    pl.kernel,
    out_type=jax.ShapeDtypeStruct((NW, OUT), jnp.float32),
    mesh=_MESH,
    scratch_types=[
        pltpu.VMEM((CR, CC), jnp.float32),
        pltpu.VMEM((CR, CC), jnp.float32),
        pltpu.VMEM((OUT,), jnp.float32),
        pltpu.SemaphoreType.DMA,
        pltpu.SemaphoreType.DMA,
    ],
)
def _sc_stage(x_hbm, part_hbm, buf0, buf1, acc, sem0, sem1):
    wid = lax.axis_index("s") * NC + lax.axis_index("c")
    row0 = BT + wid * ROWS_PER_W
    bufs = (buf0, buf1)
    sems = (sem0, sem1)

    neg = jnp.full((L,), -jnp.inf, jnp.float32)

    @pl.loop(0, OUT // L)
    def _init(p):
        acc[pl.ds(p * L, L)] = neg

    def _start(gr, q, b):
        pltpu.async_copy(
            x_hbm.at[pl.ds(row0 + gr * CR, CR), pl.ds(q * CC, CC)],
            bufs[b],
            sems[b],
        )

    def _wait(b):
        pltpu.make_async_copy(
            x_hbm.at[pl.ds(0, CR), pl.ds(0, CC)], bufs[b], sems[b]
        ).wait()

    _start(0, 0, 0)

    @pl.loop(0, NGR)
    def _main(gr):
        for q in range(NQ):
            b = q % 2
            nb = (q + 1) % 2
            if q < NQ - 1:
                _start(gr, q + 1, nb)
            else:

                @pl.when(gr + 1 < NGR)
                def _prefetch():
                    _start(gr + 1, 0, nb)

            _wait(b)
            _fold_chunk(bufs[b], acc)

    pltpu.sync_copy(acc, part_hbm.at[wid])


def _tc_body(x_ref, o_ref):
    o_ref[...] = jnp.max(x_ref[...], axis=0, keepdims=True)[None]


def _combine_body(sc_ref, tc_ref, o_ref):
    t = jnp.max(tc_ref[...].reshape(NBLK, ROW), axis=0)
    t = jnp.max(t.reshape(PER_GROUP, GROUPS, H), axis=0)
    s = jnp.max(sc_ref[...], axis=0).reshape(GROUPS, H)
    o_ref[...] = jnp.maximum(t, s).reshape(1, OUT)


def kernel(x):
    sc_parts = _sc_stage(x)
    tc_parts = pl.pallas_call(
        _tc_body,
        grid=(NBLK,),
        in_specs=[pl.BlockSpec((BR, ROW), lambda i: (i, 0))],
        out_specs=pl.BlockSpec((1, 1, ROW), lambda i: (i, 0, 0)),
        out_shape=jax.ShapeDtypeStruct((NBLK, 1, ROW), jnp.float32),
    )(x)
    return pl.pallas_call(
        _combine_body,
        out_shape=jax.ShapeDtypeStruct((1, OUT), jnp.float32),
    )(sc_parts, tc_parts)


# speedup vs baseline: 1.2538x; 1.2373x over previous
import jax
import jax.numpy as jnp
from jax.experimental import pallas as pl

H = 128
GROUPS = 16
PER_GROUP = 8
SLOTS = GROUPS * PER_GROUP
B = 4096
ROW = SLOTS * H
OUT = GROUPS * H

BR = 128
NBLK = B // BR


def _tc_body(x_ref, acc_ref, o_ref):
    i = pl.program_id(0)
    blk = jnp.max(x_ref[...].reshape(BR // 8, 8, ROW), axis=0)

    @pl.when(i == 0)
    def _init():
        acc_ref[...] = blk

    @pl.when(i > 0)
    def _accum():
        acc_ref[...] = jnp.maximum(acc_ref[...], blk)

    @pl.when(i == NBLK - 1)
    def _final():
        a = jnp.max(acc_ref[...], axis=0)
        a = jnp.max(a.reshape(PER_GROUP, GROUPS, H), axis=0)
        o_ref[...] = a.reshape(1, OUT)


def kernel(x):
    _, out = pl.pallas_call(
        _tc_body,
        grid=(NBLK,),
        in_specs=[pl.BlockSpec((BR, ROW), lambda i: (i, 0))],
        out_specs=[
            pl.BlockSpec((8, ROW), lambda i: (0, 0)),
            pl.BlockSpec((1, OUT), lambda i: (0, 0)),
        ],
        out_shape=[
            jax.ShapeDtypeStruct((8, ROW), jnp.float32),
            jax.ShapeDtypeStruct((1, OUT), jnp.float32),
        ],
    )(x)
    return out
